# baseline (speedup 1.0000x reference)
import functools

import jax
import jax.numpy as jnp
from jax import lax
from jax.experimental import pallas as pl
from jax.experimental.pallas import tpu as pltpu
from jax.experimental.pallas import tpu_sc as plsc

_K = 8192
_D = 256
_N = 4096
_KB = 512
_BETA = 0.25


def _dist_argmin_body(flat_ref, emb_ref, inds_ref, losssum_ref,
                      runval_ref, runidx_ref):
    kb = pl.program_id(0)
    nkb = pl.num_programs(0)
    flat = flat_ref[...]
    emb = emb_ref[...]
    a = jnp.sum(flat * flat, axis=1, keepdims=True)
    esq = jnp.sum(emb * emb, axis=1)
    m = lax.dot_general(flat, emb, (((1,), (1,)), ((), ())))
    dist = (a + esq[None, :]) - 2.0 * m
    bmin = jnp.min(dist, axis=1, keepdims=True)
    iota = lax.broadcasted_iota(jnp.int32, dist.shape, 1)
    bidx = jnp.min(jnp.where(dist == bmin, iota, _K), axis=1) + kb * _KB

    @pl.when(kb == 0)
    def _init():
        runval_ref[...] = bmin[:, 0]
        runidx_ref[...] = bidx

    @pl.when(kb > 0)
    def _update():
        rv = runval_ref[...]
        better = bmin[:, 0] < rv
        runval_ref[...] = jnp.where(better, bmin[:, 0], rv)
        runidx_ref[...] = jnp.where(better, bidx, runidx_ref[...])

    @pl.when(kb == nkb - 1)
    def _fin():
        inds_ref[...] = runidx_ref[...]
        losssum_ref[...] = jnp.sum(runval_ref[...]).reshape(1, 1)


def _dist_argmin(flat, emb, interpret=False):
    return pl.pallas_call(
        _dist_argmin_body,
        grid=(_K // _KB,),
        in_specs=[
            pl.BlockSpec((_N, _D), lambda k: (0, 0)),
            pl.BlockSpec((_KB, _D), lambda k: (k, 0)),
        ],
        out_specs=[
            pl.BlockSpec((_N,), lambda k: (0,)),
            pl.BlockSpec((1, 1), lambda k: (0, 0)),
        ],
        out_shape=[
            jax.ShapeDtypeStruct((_N,), jnp.int32),
            jax.ShapeDtypeStruct((1, 1), jnp.float32),
        ],
        scratch_shapes=[
            pltpu.VMEM((_N,), jnp.float32),
            pltpu.VMEM((_N,), jnp.int32),
        ],
        compiler_params=pltpu.CompilerParams(
            dimension_semantics=("arbitrary",),
        ),
        interpret=interpret,
    )(flat, emb)


@functools.cache
def _sc_gather_kernel():
    info = plsc.get_sparse_core_info()
    nw = info.num_cores * info.num_subcores
    bpw = _N // nw
    nc = info.num_cores
    mesh = plsc.VectorSubcoreMesh(core_axis_name="c", subcore_axis_name="s")

    @functools.partial(
        pl.kernel,
        out_type=jax.ShapeDtypeStruct((_N, _D), jnp.float32),
        mesh=mesh,
        scratch_types=[
            pltpu.VMEM((bpw,), jnp.int32),
            pltpu.VMEM((bpw, _D), jnp.float32),
            pltpu.SemaphoreType.DMA,
        ],
    )
    def gather_rows(table_hbm, idx_hbm, out_hbm, idx_v, rows_v, sem):
        wid = lax.axis_index("s") * nc + lax.axis_index("c")
        base = wid * bpw
        pltpu.sync_copy(idx_hbm.at[pl.ds(base, bpw)], idx_v)
        pltpu.async_copy(table_hbm.at[idx_v], rows_v, sem).wait()
        pltpu.sync_copy(rows_v, out_hbm.at[pl.ds(base, bpw)])

    return gather_rows


def kernel(latents, validation, embedding_weight):
    lat_shape = (latents.shape[0], latents.shape[2], latents.shape[3], _D)
    flat = jnp.transpose(latents, (0, 2, 3, 1)).reshape(-1, _D)
    inds, losssum = _dist_argmin(flat, embedding_weight)
    quantized = _sc_gather_kernel()(embedding_weight, inds)
    out = jnp.transpose(quantized.reshape(lat_shape), (0, 3, 1, 2))
    vq_loss = losssum[0, 0] * ((1.0 + _BETA) / (_N * _D))
    return out, vq_loss


# speedup vs baseline: 1.6253x; 1.6253x over previous
import functools

import jax
import jax.numpy as jnp
from jax import lax
from jax.experimental import pallas as pl
from jax.experimental.pallas import tpu as pltpu
from jax.experimental.pallas import tpu_sc as plsc

_K = 8192
_D = 256
_N = 4096
_KB = 512
_BETA = 0.25


_CH = 8
_NCH = _KB // _CH
_SPAN = 1024
_NSPAN = _N // _SPAN


def _dist_argmin_body(flat_ref, emb_ref, inds_ref, losssum_ref,
                      a_ref, t_ref, m_ref, rv8_ref, ri8_ref):
    kb = pl.program_id(0)
    nkb = pl.num_programs(0)
    flat = flat_ref[...]
    emb = emb_ref[...]

    @pl.when(kb == 0)
    def _init():
        a_ref[...] = jnp.sum(flat * flat, axis=1).reshape(1, _N)
        rv8_ref[...] = jnp.full((_CH, _N), jnp.inf, jnp.float32)
        ri8_ref[...] = jnp.zeros((_CH, _N), jnp.int32)

    esq = jnp.sum(emb * emb, axis=1)
    ones_kb = jnp.ones((_KB, 1), dtype=jnp.float32)
    lhs = jnp.concatenate([esq[:, None], ones_kb], axis=1)
    rhs = jnp.concatenate([jnp.ones((1, _N), jnp.float32), a_ref[...]],
                          axis=0)
    t_ref[...] = lax.dot_general(lhs, rhs, (((1,), (0,)), ((), ())))
    m_ref[...] = lax.dot_general(emb, flat, (((1,), (1,)), ((), ())))

    jbase = kb * _KB
    for sp in range(_NSPAN):
        lanes = pl.ds(sp * _SPAN, _SPAN)
        s8 = lax.broadcasted_iota(jnp.int32, (_CH, _SPAN), 0)

        def _chunk(c, carry):
            rv, ri = carry
            rows = pl.ds(c * _CH, _CH)
            d = t_ref[rows, lanes] - 2.0 * m_ref[rows, lanes]
            better = d < rv
            jc = s8 + (jbase + c * _CH)
            return (jnp.where(better, d, rv), jnp.where(better, jc, ri))

        rv, ri = lax.fori_loop(
            0, _NCH, _chunk,
            (rv8_ref[:, lanes], ri8_ref[:, lanes]), unroll=8)
        rv8_ref[:, lanes] = rv
        ri8_ref[:, lanes] = ri

    @pl.when(kb == nkb - 1)
    def _fin():
        v = rv8_ref[...]
        i = ri8_ref[...]
        for step in (4, 2, 1):
            v1, v2 = v[0:step, :], v[step:2 * step, :]
            i1, i2 = i[0:step, :], i[step:2 * step, :]
            take = (v2 < v1) | ((v2 == v1) & (i2 < i1))
            v = jnp.where(take, v2, v1)
            i = jnp.where(take, i2, i1)
        inds_ref[...] = i
        losssum_ref[...] = jnp.sum(v).reshape(1, 1)


def _dist_argmin(flat, emb, interpret=False):
    inds2, losssum = pl.pallas_call(
        _dist_argmin_body,
        grid=(_K // _KB,),
        in_specs=[
            pl.BlockSpec((_N, _D), lambda k: (0, 0)),
            pl.BlockSpec((_KB, _D), lambda k: (k, 0)),
        ],
        out_specs=[
            pl.BlockSpec((1, _N), lambda k: (0, 0)),
            pl.BlockSpec((1, 1), lambda k: (0, 0)),
        ],
        out_shape=[
            jax.ShapeDtypeStruct((1, _N), jnp.int32),
            jax.ShapeDtypeStruct((1, 1), jnp.float32),
        ],
        scratch_shapes=[
            pltpu.VMEM((1, _N), jnp.float32),
            pltpu.VMEM((_KB, _N), jnp.float32),
            pltpu.VMEM((_KB, _N), jnp.float32),
            pltpu.VMEM((_CH, _N), jnp.float32),
            pltpu.VMEM((_CH, _N), jnp.int32),
        ],
        compiler_params=pltpu.CompilerParams(
            dimension_semantics=("arbitrary",),
        ),
        interpret=interpret,
    )(flat, emb)
    return inds2.reshape(_N), losssum


@functools.cache
def _sc_gather_kernel():
    info = plsc.get_sparse_core_info()
    nw = info.num_cores * info.num_subcores
    bpw = _N // nw
    nc = info.num_cores
    mesh = plsc.VectorSubcoreMesh(core_axis_name="c", subcore_axis_name="s")

    @functools.partial(
        pl.kernel,
        out_type=jax.ShapeDtypeStruct((_N, _D), jnp.float32),
        mesh=mesh,
        scratch_types=[
            pltpu.VMEM((bpw,), jnp.int32),
            pltpu.VMEM((bpw, _D), jnp.float32),
            pltpu.SemaphoreType.DMA,
        ],
    )
    def gather_rows(table_hbm, idx_hbm, out_hbm, idx_v, rows_v, sem):
        wid = lax.axis_index("s") * nc + lax.axis_index("c")
        base = wid * bpw
        pltpu.sync_copy(idx_hbm.at[pl.ds(base, bpw)], idx_v)
        pltpu.async_copy(table_hbm.at[idx_v], rows_v, sem).wait()
        pltpu.sync_copy(rows_v, out_hbm.at[pl.ds(base, bpw)])

    return gather_rows


def kernel(latents, validation, embedding_weight):
    lat_shape = (latents.shape[0], latents.shape[2], latents.shape[3], _D)
    flat = jnp.transpose(latents, (0, 2, 3, 1)).reshape(-1, _D)
    inds, losssum = _dist_argmin(flat, embedding_weight)
    quantized = _sc_gather_kernel()(embedding_weight, inds)
    out = jnp.transpose(quantized.reshape(lat_shape), (0, 3, 1, 2))
    vq_loss = losssum[0, 0] * ((1.0 + _BETA) / (_N * _D))
    return out, vq_loss


# speedup vs baseline: 1.6461x; 1.0128x over previous
import functools

import jax
import jax.numpy as jnp
from jax import lax
from jax.experimental import pallas as pl
from jax.experimental.pallas import tpu as pltpu
from jax.experimental.pallas import tpu_sc as plsc

_K = 8192
_D = 256
_N = 4096
_KB = 512
_BETA = 0.25


_CH = 8
_NCH = _KB // _CH
_SPAN = 1024
_NSPAN = _N // _SPAN


_KPAD = 128


def _dist_argmin_body(flat_ref, emb_ref, inds_ref, losssum_ref,
                      rhs_ref, t_ref, m_ref, rv8_ref, ri8_ref):
    kb = pl.program_id(0)
    nkb = pl.num_programs(0)
    flat = flat_ref[...]
    emb = emb_ref[...]

    @pl.when(kb == 0)
    def _init():
        a = jnp.sum(flat * flat, axis=1).reshape(1, _N)
        rhs_ref[...] = jnp.concatenate(
            [jnp.ones((1, _N), jnp.float32), a,
             jnp.zeros((_KPAD - 2, _N), jnp.float32)], axis=0)
        rv8_ref[...] = jnp.full((_CH, _N), jnp.inf, jnp.float32)
        ri8_ref[...] = jnp.zeros((_CH, _N), jnp.int32)

    esq = jnp.sum(emb * emb, axis=1)
    lhs = jnp.concatenate(
        [esq[:, None], jnp.ones((_KB, 1), jnp.float32),
         jnp.zeros((_KB, _KPAD - 2), jnp.float32)], axis=1)
    t_ref[...] = lax.dot_general(lhs, rhs_ref[...], (((1,), (0,)), ((), ())))
    m_ref[...] = lax.dot_general(emb, flat, (((1,), (1,)), ((), ())))

    jbase = kb * _KB
    for sp in range(_NSPAN):
        lanes = pl.ds(sp * _SPAN, _SPAN)
        s8 = lax.broadcasted_iota(jnp.int32, (_CH, _SPAN), 0)

        def _chunk(c, carry):
            rv, ri = carry
            rows = pl.ds(pl.multiple_of(c * _CH, _CH), _CH)
            d = t_ref[rows, lanes] - 2.0 * m_ref[rows, lanes]
            better = d < rv
            jc = s8 + (jbase + c * _CH)
            return (jnp.where(better, d, rv), jnp.where(better, jc, ri))

        rv, ri = lax.fori_loop(
            0, _NCH, _chunk,
            (rv8_ref[:, lanes], ri8_ref[:, lanes]), unroll=8)
        rv8_ref[:, lanes] = rv
        ri8_ref[:, lanes] = ri

    @pl.when(kb == nkb - 1)
    def _fin():
        v = rv8_ref[...]
        i = ri8_ref[...]
        for step in (4, 2, 1):
            v1, v2 = v[0:step, :], v[step:2 * step, :]
            i1, i2 = i[0:step, :], i[step:2 * step, :]
            take = (v2 < v1) | ((v2 == v1) & (i2 < i1))
            v = jnp.where(take, v2, v1)
            i = jnp.where(take, i2, i1)
        inds_ref[...] = i
        losssum_ref[...] = jnp.sum(v).reshape(1, 1)


def _dist_argmin(flat, emb, interpret=False):
    inds2, losssum = pl.pallas_call(
        _dist_argmin_body,
        grid=(_K // _KB,),
        in_specs=[
            pl.BlockSpec((_N, _D), lambda k: (0, 0)),
            pl.BlockSpec((_KB, _D), lambda k: (k, 0)),
        ],
        out_specs=[
            pl.BlockSpec((1, _N), lambda k: (0, 0)),
            pl.BlockSpec((1, 1), lambda k: (0, 0)),
        ],
        out_shape=[
            jax.ShapeDtypeStruct((1, _N), jnp.int32),
            jax.ShapeDtypeStruct((1, 1), jnp.float32),
        ],
        scratch_shapes=[
            pltpu.VMEM((_KPAD, _N), jnp.float32),
            pltpu.VMEM((_KB, _N), jnp.float32),
            pltpu.VMEM((_KB, _N), jnp.float32),
            pltpu.VMEM((_CH, _N), jnp.float32),
            pltpu.VMEM((_CH, _N), jnp.int32),
        ],
        compiler_params=pltpu.CompilerParams(
            dimension_semantics=("arbitrary",),
        ),
        interpret=interpret,
    )(flat, emb)
    return inds2.reshape(_N), losssum


@functools.cache
def _sc_gather_kernel():
    info = plsc.get_sparse_core_info()
    nw = info.num_cores * info.num_subcores
    bpw = _N // nw
    nc = info.num_cores
    mesh = plsc.VectorSubcoreMesh(core_axis_name="c", subcore_axis_name="s")

    @functools.partial(
        pl.kernel,
        out_type=jax.ShapeDtypeStruct((_N, _D), jnp.float32),
        mesh=mesh,
        scratch_types=[
            pltpu.VMEM((bpw,), jnp.int32),
            pltpu.VMEM((bpw, _D), jnp.float32),
            pltpu.SemaphoreType.DMA,
        ],
    )
    def gather_rows(table_hbm, idx_hbm, out_hbm, idx_v, rows_v, sem):
        wid = lax.axis_index("s") * nc + lax.axis_index("c")
        base = wid * bpw
        pltpu.sync_copy(idx_hbm.at[pl.ds(base, bpw)], idx_v)
        pltpu.async_copy(table_hbm.at[idx_v], rows_v, sem).wait()
        pltpu.sync_copy(rows_v, out_hbm.at[pl.ds(base, bpw)])

    return gather_rows


def kernel(latents, validation, embedding_weight):
    lat_shape = (latents.shape[0], latents.shape[2], latents.shape[3], _D)
    flat = jnp.transpose(latents, (0, 2, 3, 1)).reshape(-1, _D)
    inds, losssum = _dist_argmin(flat, embedding_weight)
    quantized = _sc_gather_kernel()(embedding_weight, inds)
    out = jnp.transpose(quantized.reshape(lat_shape), (0, 3, 1, 2))
    vq_loss = losssum[0, 0] * ((1.0 + _BETA) / (_N * _D))
    return out, vq_loss
